# baseline (speedup 1.0000x reference)
import jax
import jax.numpy as jnp
from jax import lax
from jax.experimental import pallas as pl
from jax.experimental.pallas import tpu as pltpu

ALPHA = 0.2
BM = 512
_DEFAULT = jax.lax.Precision.DEFAULT


def _body(adj_ref, x_ref, w0_ref, w1t_ref, b0_ref, b1_ref, o_ref, z1_ref):
    p = pl.program_id(0)
    i = pl.program_id(1)

    @pl.when(p == 0)
    def _layer1():
        t = jnp.dot(
            adj_ref[...], x_ref[...],
            preferred_element_type=jnp.float32, precision=_DEFAULT,
        )
        h = jnp.dot(
            t, w0_ref[...],
            preferred_element_type=jnp.float32, precision=_DEFAULT,
        )
        h = h + b0_ref[...]
        h = jnp.where(h >= 0, h, ALPHA * h)
        z1_ref[pl.ds(i * BM, BM), :] = lax.dot_general(
            h, w1t_ref[...],
            (((1,), (1,)), ((), ())),
            preferred_element_type=jnp.float32, precision=_DEFAULT,
        )

    @pl.when(p == 1)
    def _layer2():
        n = x_ref.shape[0]
        h = jnp.dot(
            adj_ref[...], z1_ref[pl.ds(0, n), :],
            preferred_element_type=jnp.float32, precision=_DEFAULT,
        )
        h = h + b1_ref[...]
        m = jnp.max(h, axis=1, keepdims=True)
        e = jnp.exp(h - m)
        s = jnp.sum(e, axis=1, keepdims=True)
        res = (h - m) - jnp.log(s)
        o_ref[...] = res.T


def kernel(x, edge_feats, adj, W0, b0, W1, b1):
    del edge_feats
    n, nfeat = x.shape
    nhid = W0.shape[1]
    nclass = W1.shape[1]
    b0r = b0.reshape(1, nhid)
    b1r = b1.reshape(1, nclass)
    w1t = W1.T

    nblocks = pl.cdiv(n, BM)
    npad = nblocks * BM

    out_t = pl.pallas_call(
        _body,
        grid=(2, nblocks),
        in_specs=[
            pl.BlockSpec((BM, n), lambda p, i: (i, 0)),
            pl.BlockSpec((n, nfeat), lambda p, i: (0, 0)),
            pl.BlockSpec((nfeat, nhid), lambda p, i: (0, 0)),
            pl.BlockSpec((nclass, nhid), lambda p, i: (0, 0)),
            pl.BlockSpec((1, nhid), lambda p, i: (0, 0)),
            pl.BlockSpec((1, nclass), lambda p, i: (0, 0)),
        ],
        out_specs=pl.BlockSpec((nclass, BM), lambda p, i: (1 - p, p * i)),
        out_shape=jax.ShapeDtypeStruct((2 * nclass, n), jnp.float32),
        scratch_shapes=[
            pltpu.VMEM((npad, nclass), jnp.float32),
        ],
        compiler_params=pltpu.CompilerParams(
            dimension_semantics=("arbitrary", "arbitrary")),
    )(adj, x, W0, w1t, b0r, b1r)

    return out_t[:nclass].T


# speedup vs baseline: 1.0021x; 1.0021x over previous
import jax
import jax.numpy as jnp
from jax import lax
from jax.experimental import pallas as pl
from jax.experimental.pallas import tpu as pltpu

ALPHA = 0.2
BM = 512
_DEFAULT = jax.lax.Precision.DEFAULT


def _body(adj_ref, x_ref, w0_ref, w1t_ref, b0_ref, b1_ref, o_ref,
          z1_ref, w0bf_ref, w1tbf_ref):
    p = pl.program_id(0)
    i = pl.program_id(1)

    @pl.when(jnp.logical_and(p == 0, i == 0))
    def _init():
        w0bf_ref[...] = w0_ref[...].astype(jnp.bfloat16)
        w1tbf_ref[...] = w1t_ref[...].astype(jnp.bfloat16)

    @pl.when(p == 0)
    def _layer1():
        t = jnp.dot(
            adj_ref[...], x_ref[...],
            preferred_element_type=jnp.float32, precision=_DEFAULT,
        )
        h = jnp.dot(
            t.astype(jnp.bfloat16), w0bf_ref[...],
            preferred_element_type=jnp.float32,
        )
        h = h + b0_ref[...]
        h = jnp.where(h >= 0, h, ALPHA * h)
        z1 = lax.dot_general(
            h.astype(jnp.bfloat16), w1tbf_ref[...],
            (((1,), (1,)), ((), ())),
            preferred_element_type=jnp.float32,
        )
        z1_ref[pl.ds(i * BM, BM), :] = z1.astype(jnp.bfloat16)

    @pl.when(p == 1)
    def _layer2():
        n = x_ref.shape[0]
        h = jnp.dot(
            adj_ref[...], z1_ref[pl.ds(0, n), :].astype(jnp.float32),
            preferred_element_type=jnp.float32, precision=_DEFAULT,
        )
        h = h + b1_ref[...]
        m = jnp.max(h, axis=1, keepdims=True)
        e = jnp.exp(h - m)
        s = jnp.sum(e, axis=1, keepdims=True)
        res = (h - m) - jnp.log(s)
        o_ref[...] = res.T


def kernel(x, edge_feats, adj, W0, b0, W1, b1):
    del edge_feats
    n, nfeat = x.shape
    nhid = W0.shape[1]
    nclass = W1.shape[1]
    b0r = b0.reshape(1, nhid)
    b1r = b1.reshape(1, nclass)
    w1t = W1.T

    nblocks = pl.cdiv(n, BM)
    npad = nblocks * BM

    out_t = pl.pallas_call(
        _body,
        grid=(2, nblocks),
        in_specs=[
            pl.BlockSpec((BM, n), lambda p, i: (i, 0)),
            pl.BlockSpec((n, nfeat), lambda p, i: (0, 0)),
            pl.BlockSpec((nfeat, nhid), lambda p, i: (0, 0)),
            pl.BlockSpec((nclass, nhid), lambda p, i: (0, 0)),
            pl.BlockSpec((1, nhid), lambda p, i: (0, 0)),
            pl.BlockSpec((1, nclass), lambda p, i: (0, 0)),
        ],
        out_specs=pl.BlockSpec((nclass, BM), lambda p, i: (1 - p, p * i)),
        out_shape=jax.ShapeDtypeStruct((2 * nclass, n), jnp.float32),
        scratch_shapes=[
            pltpu.VMEM((npad, nclass), jnp.bfloat16),
            pltpu.VMEM((nfeat, nhid), jnp.bfloat16),
            pltpu.VMEM((nclass, nhid), jnp.bfloat16),
        ],
        compiler_params=pltpu.CompilerParams(
            dimension_semantics=("arbitrary", "arbitrary")),
    )(adj, x, W0, w1t, b0r, b1r)

    return out_t[:nclass].T
